# Initial kernel scaffold; baseline (speedup 1.0000x reference)
import jax
import jax.numpy as jnp
from jax.experimental import pallas as pl


def kernel(x, edge_index, W1, b1, W2, b2, W3, b3, C1, cb1, C2, cb2):
    raise NotImplementedError("write your pallas kernel here")


# speedup vs baseline: 3.8966x; 3.8966x over previous
import functools

import jax
import jax.numpy as jnp
from jax import lax
from jax.experimental import pallas as pl
from jax.experimental.pallas import tpu as pltpu
from jax.experimental.pallas import tpu_sc as plsc

N = 10000
NP = 10240
F = 256
H = 512
CW = 128
NCHUNK = 4
NC = 2
NS = 16
NW = NC * NS
E = 160000
ET = E // NW
EB = 128
NB = 40
ETP = EB * NB
TRASH = N
RPT = NP // NS
BN = 512
G = NP // BN


def _sc_mesh():
    return plsc.VectorSubcoreMesh(core_axis_name="c", subcore_axis_name="s")


def _sc_degree(dstp, ones_rows, zeros_rows):
    @functools.partial(
        pl.kernel,
        mesh=_sc_mesh(),
        out_type=jax.ShapeDtypeStruct((NC * NP, CW), jnp.float32),
        scratch_types=[
            pltpu.VMEM((NB, EB), jnp.int32),
            pltpu.VMEM((EB, CW), jnp.float32),
            pltpu.VMEM_SHARED((NP, CW), jnp.float32),
        ],
    )
    def k(dst_hbm, ones_hbm, zeros_hbm, out_hbm, dst_v, ones_v, acc):
        cid = lax.axis_index("c")
        sid = lax.axis_index("s")
        w = cid * NS + sid
        pltpu.sync_copy(dst_hbm.at[w], dst_v)
        pltpu.sync_copy(ones_hbm, ones_v)

        pltpu.sync_copy(zeros_hbm, acc.at[pl.ds(sid * RPT, RPT)])
        plsc.subcore_barrier()

        def body(j, carry):
            pltpu.sync_copy(ones_v, acc.at[dst_v.at[j]], add=True)
            return carry

        lax.fori_loop(0, NB, body, 0)
        plsc.subcore_barrier()
        pltpu.sync_copy(
            acc.at[pl.ds(sid * RPT, RPT)],
            out_hbm.at[pl.ds(cid * NP + sid * RPT, RPT)],
        )

    return k(dstp, ones_rows, zeros_rows)


def _sc_aggregate(srcp, dstp, tables, zeros_rows):
    @functools.partial(
        pl.kernel,
        mesh=_sc_mesh(),
        out_type=[jax.ShapeDtypeStruct((NC * NP, CW), jnp.float32)] * NCHUNK,
        scratch_types=[
            pltpu.VMEM((NB, EB), jnp.int32),
            pltpu.VMEM((NB, EB), jnp.int32),
            pltpu.VMEM((EB, CW), jnp.float32),
            pltpu.VMEM_SHARED((NP, CW), jnp.float32),
            pltpu.SemaphoreType.DMA,
        ],
    )
    def k(src_hbm, dst_hbm, t0, t1, t2, t3, zeros_hbm,
          o0, o1, o2, o3,
          src_v, dst_v, gbuf, acc, sem):
        cid = lax.axis_index("c")
        sid = lax.axis_index("s")
        w = cid * NS + sid
        pltpu.sync_copy(src_hbm.at[w], src_v)
        pltpu.sync_copy(dst_hbm.at[w], dst_v)

        tabs = (t0, t1, t2, t3)
        outs = (o0, o1, o2, o3)
        for tab, out in zip(tabs, outs):
            pltpu.sync_copy(zeros_hbm, acc.at[pl.ds(sid * RPT, RPT)])
            plsc.subcore_barrier()

            def body(j, carry, tab=tab):
                pltpu.async_copy(tab.at[src_v.at[j]], gbuf, sem).wait()
                pltpu.sync_copy(gbuf, acc.at[dst_v.at[j]], add=True)
                return carry

            lax.fori_loop(0, NB, body, 0)
            plsc.subcore_barrier()
            pltpu.sync_copy(
                acc.at[pl.ds(sid * RPT, RPT)],
                out.at[pl.ds(cid * NP + sid * RPT, RPT)],
            )
            plsc.subcore_barrier()

    return k(srcp, dstp, *tables, zeros_rows)


def _dinv_cat(d0, d1):
    deg = d0 + d1 + 1.0
    dinv = lax.rsqrt(deg)
    return jnp.concatenate([dinv] * NCHUNK, axis=1)


def _split_store(pp, outs):
    for c, o in enumerate(outs):
        o[...] = pp[:, c * CW:(c + 1) * CW]


def _tc_first(xp, deg2, W1):
    def body(x_ref, d0_ref, d1_ref, w_ref, *outs):
        dinv8 = _dinv_cat(d0_ref[...], d1_ref[...])
        p = lax.dot_general(x_ref[...], w_ref[...], (((1,), (1,)), ((), ())),
                            preferred_element_type=jnp.float32)
        _split_store(p * dinv8, outs)

    return pl.pallas_call(
        body,
        grid=(G,),
        in_specs=[
            pl.BlockSpec((BN, F), lambda i: (i, 0)),
            pl.BlockSpec((BN, CW), lambda i: (i, 0)),
            pl.BlockSpec((BN, CW), lambda i: (i + G, 0)),
            pl.BlockSpec((H, F), lambda i: (0, 0)),
        ],
        out_specs=[pl.BlockSpec((BN, CW), lambda i: (i, 0))] * NCHUNK,
        out_shape=[jax.ShapeDtypeStruct((NP, CW), jnp.float32)] * NCHUNK,
    )(xp, deg2, deg2, W1)


def _relu_layer(a0s, a1s, ps, d0_ref, d1_ref, b_ref):
    dinv8 = _dinv_cat(d0_ref[...], d1_ref[...])
    cat = jnp.concatenate(
        [a0s[c][...] + a1s[c][...] + ps[c][...] for c in range(NCHUNK)], axis=1)
    brow = jnp.concatenate(
        [b_ref[c:c + 1, :] for c in range(NCHUNK)], axis=1)
    return jnp.maximum(dinv8 * cat + brow, 0.0), dinv8


def _tc_mid(aggs, ps, deg2, b_pad, W):
    def body(*refs):
        a0s, a1s, pcs = (refs[0:NCHUNK], refs[NCHUNK:2 * NCHUNK],
                         refs[2 * NCHUNK:3 * NCHUNK])
        d0_ref, d1_ref, b_ref, w_ref = refs[3 * NCHUNK:3 * NCHUNK + 4]
        outs = refs[3 * NCHUNK + 4:]
        h, dinv8 = _relu_layer(a0s, a1s, pcs, d0_ref, d1_ref, b_ref)
        p = lax.dot_general(h, w_ref[...], (((1,), (1,)), ((), ())),
                            preferred_element_type=jnp.float32)
        _split_store(p * dinv8, outs)

    blk = pl.BlockSpec((BN, CW), lambda i: (i, 0))
    blk1 = pl.BlockSpec((BN, CW), lambda i: (i + G, 0))
    args = list(aggs) + list(aggs) + list(ps) + [deg2, deg2, b_pad, W]
    return pl.pallas_call(
        body,
        grid=(G,),
        in_specs=[blk] * NCHUNK + [blk1] * NCHUNK + [blk] * NCHUNK +
                 [blk, blk1,
                  pl.BlockSpec((8, CW), lambda i: (0, 0)),
                  pl.BlockSpec((H, H), lambda i: (0, 0))],
        out_specs=[pl.BlockSpec((BN, CW), lambda i: (i, 0))] * NCHUNK,
        out_shape=[jax.ShapeDtypeStruct((NP, CW), jnp.float32)] * NCHUNK,
    )(*args)


def _tc_head(aggs, ps, deg2, b_pad, c1t_pad, cb1_pad, c2t_pad, cb2_pad):
    def body(*refs):
        a0s, a1s, pcs = (refs[0:NCHUNK], refs[NCHUNK:2 * NCHUNK],
                         refs[2 * NCHUNK:3 * NCHUNK])
        d0_ref, d1_ref, b_ref = refs[3 * NCHUNK:3 * NCHUNK + 3]
        c1t_ref, cb1_ref, c2t_ref, cb2_ref = refs[3 * NCHUNK + 3:3 * NCHUNK + 7]
        out_ref = refs[3 * NCHUNK + 7]
        acc = refs[3 * NCHUNK + 8]
        i = pl.program_id(0)
        h, _ = _relu_layer(a0s, a1s, pcs, d0_ref, d1_ref, b_ref)
        rows = i * BN + lax.broadcasted_iota(jnp.int32, (BN, 1), 0)
        h = jnp.where(rows < N, h, 0.0)

        @pl.when(i == 0)
        def _():
            acc[...] = jnp.zeros_like(acc)

        acc[...] += h

        @pl.when(i == G - 1)
        def _():
            g = jnp.sum(acc[...], axis=0, keepdims=True) * (1.0 / N)
            z = jnp.maximum(
                lax.dot_general(g, c1t_ref[...], (((1,), (0,)), ((), ())),
                                preferred_element_type=jnp.float32)
                + cb1_ref[0:1, :], 0.0)
            o = lax.dot_general(z, c2t_ref[...], (((1,), (0,)), ((), ())),
                                preferred_element_type=jnp.float32)
            o = o + cb2_ref[0:1, :]
            out_ref[...] = jnp.broadcast_to(o[0:1, 0:1], (8, 128))

    blk = pl.BlockSpec((BN, CW), lambda i: (i, 0))
    blk1 = pl.BlockSpec((BN, CW), lambda i: (i + G, 0))
    full = lambda shape: pl.BlockSpec(shape, lambda i: tuple(0 for _ in shape))
    args = list(aggs) + list(aggs) + list(ps) + [deg2, deg2, b_pad,
            c1t_pad, cb1_pad, c2t_pad, cb2_pad]
    return pl.pallas_call(
        body,
        grid=(G,),
        in_specs=[blk] * NCHUNK + [blk1] * NCHUNK + [blk] * NCHUNK +
                 [blk, blk1, full((8, CW)),
                  full((H, 128)), full((8, 128)), full((128, 128)),
                  full((8, 128))],
        out_specs=pl.BlockSpec((8, 128), lambda i: (0, 0)),
        out_shape=jax.ShapeDtypeStruct((8, 128), jnp.float32),
        scratch_shapes=[pltpu.VMEM((BN, H), jnp.float32)],
    )(*args)


def kernel(x, edge_index, W1, b1, W2, b2, W3, b3, C1, cb1, C2, cb2):
    f32 = jnp.float32
    xp = jnp.zeros((NP, F), f32).at[:N].set(x)
    src = edge_index[0].reshape(NW, ET)
    dst = edge_index[1].reshape(NW, ET)
    srcp = jnp.pad(src, ((0, 0), (0, ETP - ET))).reshape(NW, NB, EB)
    dstp = jnp.pad(dst, ((0, 0), (0, ETP - ET)),
                   constant_values=TRASH).reshape(NW, NB, EB)
    ones_rows = jnp.ones((EB, CW), f32)
    zeros_rows = jnp.zeros((RPT, CW), f32)

    def pad_bias(b):
        return b.reshape(NCHUNK, CW)

    b1p, b2p, b3p = pad_bias(b1), pad_bias(b2), pad_bias(b3)
    c1t_pad = jnp.zeros((H, 128), f32).at[:, 0:64].set(C1.T)
    cb1_pad = jnp.zeros((8, 128), f32).at[0, 0:64].set(cb1)
    c2t_pad = jnp.zeros((128, 128), f32).at[0:64, 0:1].set(C2.T)
    cb2_pad = jnp.full((8, 128), cb2[0], f32)

    deg2 = _sc_degree(dstp, ones_rows, zeros_rows)
    p1 = _tc_first(xp, deg2, W1)
    a1 = _sc_aggregate(srcp, dstp, p1, zeros_rows)
    p2 = _tc_mid(a1, p1, deg2, b1p, W2)
    a2 = _sc_aggregate(srcp, dstp, p2, zeros_rows)
    p3 = _tc_mid(a2, p2, deg2, b2p, W3)
    a3 = _sc_aggregate(srcp, dstp, p3, zeros_rows)
    out = _tc_head(a3, p3, deg2, b3p, c1t_pad, cb1_pad, c2t_pad, cb2_pad)
    return out[0, 0].reshape(1)


# speedup vs baseline: 4.1817x; 1.0732x over previous
import functools

import jax
import jax.numpy as jnp
from jax import lax
from jax.experimental import pallas as pl
from jax.experimental.pallas import tpu as pltpu
from jax.experimental.pallas import tpu_sc as plsc

N = 10000
NP = 10240
F = 256
H = 512
CW = 128
NCHUNK = 4
NC = 2
NS = 16
NW = NC * NS
E = 160000
ET = E // NW
EB = 128
NB = 40
ETP = EB * NB
TRASH = N
RPT = NP // NS
BN = 512
G = NP // BN


def _sc_mesh():
    return plsc.VectorSubcoreMesh(core_axis_name="c", subcore_axis_name="s")


def _sc_degree(dstp, ones_rows, zeros_rows):
    @functools.partial(
        pl.kernel,
        mesh=_sc_mesh(),
        out_type=jax.ShapeDtypeStruct((NC * NP, CW), jnp.float32),
        scratch_types=[
            pltpu.VMEM((NB, EB), jnp.int32),
            pltpu.VMEM((EB, CW), jnp.float32),
            pltpu.VMEM_SHARED((NP, CW), jnp.float32),
            pltpu.SemaphoreType.DMA,
        ],
    )
    def k(dst_hbm, ones_hbm, zeros_hbm, out_hbm, dst_v, ones_v, acc, sem):
        cid = lax.axis_index("c")
        sid = lax.axis_index("s")
        w = cid * NS + sid
        pltpu.sync_copy(dst_hbm.at[w], dst_v)
        pltpu.sync_copy(ones_hbm, ones_v)

        pltpu.sync_copy(zeros_hbm, acc.at[pl.ds(sid * RPT, RPT)])
        plsc.subcore_barrier()

        def group(g, carry):
            for u in range(8):
                pltpu.async_copy(ones_v, acc.at[dst_v.at[g * 8 + u]], sem,
                                 add=True)
            for u in range(8):
                pltpu.make_async_copy(ones_v, acc.at[dst_v.at[g * 8 + u]],
                                      sem).wait()
            return carry

        lax.fori_loop(0, NB // 8, group, 0)
        plsc.subcore_barrier()
        pltpu.sync_copy(
            acc.at[pl.ds(sid * RPT, RPT)],
            out_hbm.at[pl.ds(cid * NP + sid * RPT, RPT)],
        )

    return k(dstp, ones_rows, zeros_rows)


def _sc_aggregate(srcp, dstp, tables, zeros_rows):
    @functools.partial(
        pl.kernel,
        mesh=_sc_mesh(),
        out_type=[jax.ShapeDtypeStruct((NC * NP, CW), jnp.float32)] * NCHUNK,
        scratch_types=[
            pltpu.VMEM((NB, EB), jnp.int32),
            pltpu.VMEM((NB, EB), jnp.int32),
            pltpu.VMEM((EB, CW), jnp.float32),
            pltpu.VMEM((EB, CW), jnp.float32),
            pltpu.VMEM_SHARED((NP, CW), jnp.float32),
            pltpu.SemaphoreType.DMA,
            pltpu.SemaphoreType.DMA,
            pltpu.SemaphoreType.DMA,
            pltpu.SemaphoreType.DMA,
        ],
    )
    def k(src_hbm, dst_hbm, t0, t1, t2, t3, zeros_hbm,
          o0, o1, o2, o3,
          src_v, dst_v, gb0, gb1, acc, sg0, sg1, ss0, ss1):
        cid = lax.axis_index("c")
        sid = lax.axis_index("s")
        w = cid * NS + sid
        pltpu.sync_copy(src_hbm.at[w], src_v)
        pltpu.sync_copy(dst_hbm.at[w], dst_v)

        tabs = (t0, t1, t2, t3)
        outs = (o0, o1, o2, o3)
        for tab, out in zip(tabs, outs):
            pltpu.sync_copy(zeros_hbm, acc.at[pl.ds(sid * RPT, RPT)])
            plsc.subcore_barrier()

            def gather(j, gb, sg, tab=tab):
                return pltpu.async_copy(tab.at[src_v.at[j]], gb, sg)

            def gwait(j, gb, sg, tab=tab):
                pltpu.make_async_copy(tab.at[src_v.at[j]], gb, sg).wait()

            def scat(j, gb, ss):
                return pltpu.async_copy(gb, acc.at[dst_v.at[j]], ss, add=True)

            def swait(j, gb, ss):
                pltpu.make_async_copy(gb, acc.at[dst_v.at[j]], ss).wait()

            gather(0, gb0, sg0)
            gather(1, gb1, sg1)

            def pair(jj, carry):
                j0, j1 = 2 * jj, 2 * jj + 1
                gwait(j0, gb0, sg0)
                scat(j0, gb0, ss0)
                gwait(j1, gb1, sg1)
                scat(j1, gb1, ss1)
                swait(j0, gb0, ss0)
                gather(j0 + 2, gb0, sg0)
                swait(j1, gb1, ss1)
                gather(j1 + 2, gb1, sg1)
                return carry

            lax.fori_loop(0, NB // 2 - 1, pair, 0)
            j0, j1 = NB - 2, NB - 1
            gwait(j0, gb0, sg0)
            scat(j0, gb0, ss0)
            gwait(j1, gb1, sg1)
            scat(j1, gb1, ss1)
            swait(j0, gb0, ss0)
            swait(j1, gb1, ss1)
            plsc.subcore_barrier()
            pltpu.sync_copy(
                acc.at[pl.ds(sid * RPT, RPT)],
                out.at[pl.ds(cid * NP + sid * RPT, RPT)],
            )
            plsc.subcore_barrier()

    return k(srcp, dstp, *tables, zeros_rows)


def _dinv_cat(d0, d1):
    deg = d0 + d1 + 1.0
    dinv = lax.rsqrt(deg)
    return jnp.concatenate([dinv] * NCHUNK, axis=1)


def _split_store(pp, outs):
    for c, o in enumerate(outs):
        o[...] = pp[:, c * CW:(c + 1) * CW]


def _tc_first(xp, deg2, W1):
    def body(x_ref, d0_ref, d1_ref, w_ref, *outs):
        dinv8 = _dinv_cat(d0_ref[...], d1_ref[...])
        p = lax.dot_general(x_ref[...], w_ref[...], (((1,), (1,)), ((), ())),
                            preferred_element_type=jnp.float32)
        _split_store(p * dinv8, outs)

    return pl.pallas_call(
        body,
        grid=(G,),
        in_specs=[
            pl.BlockSpec((BN, F), lambda i: (i, 0)),
            pl.BlockSpec((BN, CW), lambda i: (i, 0)),
            pl.BlockSpec((BN, CW), lambda i: (i + G, 0)),
            pl.BlockSpec((H, F), lambda i: (0, 0)),
        ],
        out_specs=[pl.BlockSpec((BN, CW), lambda i: (i, 0))] * NCHUNK,
        out_shape=[jax.ShapeDtypeStruct((NP, CW), jnp.float32)] * NCHUNK,
    )(xp, deg2, deg2, W1)


def _relu_layer(a0s, a1s, ps, d0_ref, d1_ref, b_ref):
    dinv8 = _dinv_cat(d0_ref[...], d1_ref[...])
    cat = jnp.concatenate(
        [a0s[c][...] + a1s[c][...] + ps[c][...] for c in range(NCHUNK)], axis=1)
    brow = jnp.concatenate(
        [b_ref[c:c + 1, :] for c in range(NCHUNK)], axis=1)
    return jnp.maximum(dinv8 * cat + brow, 0.0), dinv8


def _tc_mid(aggs, ps, deg2, b_pad, W):
    def body(*refs):
        a0s, a1s, pcs = (refs[0:NCHUNK], refs[NCHUNK:2 * NCHUNK],
                         refs[2 * NCHUNK:3 * NCHUNK])
        d0_ref, d1_ref, b_ref, w_ref = refs[3 * NCHUNK:3 * NCHUNK + 4]
        outs = refs[3 * NCHUNK + 4:]
        h, dinv8 = _relu_layer(a0s, a1s, pcs, d0_ref, d1_ref, b_ref)
        p = lax.dot_general(h, w_ref[...], (((1,), (1,)), ((), ())),
                            preferred_element_type=jnp.float32)
        _split_store(p * dinv8, outs)

    blk = pl.BlockSpec((BN, CW), lambda i: (i, 0))
    blk1 = pl.BlockSpec((BN, CW), lambda i: (i + G, 0))
    args = list(aggs) + list(aggs) + list(ps) + [deg2, deg2, b_pad, W]
    return pl.pallas_call(
        body,
        grid=(G,),
        in_specs=[blk] * NCHUNK + [blk1] * NCHUNK + [blk] * NCHUNK +
                 [blk, blk1,
                  pl.BlockSpec((8, CW), lambda i: (0, 0)),
                  pl.BlockSpec((H, H), lambda i: (0, 0))],
        out_specs=[pl.BlockSpec((BN, CW), lambda i: (i, 0))] * NCHUNK,
        out_shape=[jax.ShapeDtypeStruct((NP, CW), jnp.float32)] * NCHUNK,
    )(*args)


def _tc_head(aggs, ps, deg2, b_pad, c1t_pad, cb1_pad, c2t_pad, cb2_pad):
    def body(*refs):
        a0s, a1s, pcs = (refs[0:NCHUNK], refs[NCHUNK:2 * NCHUNK],
                         refs[2 * NCHUNK:3 * NCHUNK])
        d0_ref, d1_ref, b_ref = refs[3 * NCHUNK:3 * NCHUNK + 3]
        c1t_ref, cb1_ref, c2t_ref, cb2_ref = refs[3 * NCHUNK + 3:3 * NCHUNK + 7]
        out_ref = refs[3 * NCHUNK + 7]
        acc = refs[3 * NCHUNK + 8]
        i = pl.program_id(0)
        h, _ = _relu_layer(a0s, a1s, pcs, d0_ref, d1_ref, b_ref)
        rows = i * BN + lax.broadcasted_iota(jnp.int32, (BN, 1), 0)
        h = jnp.where(rows < N, h, 0.0)

        @pl.when(i == 0)
        def _():
            acc[...] = jnp.zeros_like(acc)

        acc[...] += h

        @pl.when(i == G - 1)
        def _():
            g = jnp.sum(acc[...], axis=0, keepdims=True) * (1.0 / N)
            z = jnp.maximum(
                lax.dot_general(g, c1t_ref[...], (((1,), (0,)), ((), ())),
                                preferred_element_type=jnp.float32)
                + cb1_ref[0:1, :], 0.0)
            o = lax.dot_general(z, c2t_ref[...], (((1,), (0,)), ((), ())),
                                preferred_element_type=jnp.float32)
            o = o + cb2_ref[0:1, :]
            out_ref[...] = jnp.broadcast_to(o[0:1, 0:1], (8, 128))

    blk = pl.BlockSpec((BN, CW), lambda i: (i, 0))
    blk1 = pl.BlockSpec((BN, CW), lambda i: (i + G, 0))
    full = lambda shape: pl.BlockSpec(shape, lambda i: tuple(0 for _ in shape))
    args = list(aggs) + list(aggs) + list(ps) + [deg2, deg2, b_pad,
            c1t_pad, cb1_pad, c2t_pad, cb2_pad]
    return pl.pallas_call(
        body,
        grid=(G,),
        in_specs=[blk] * NCHUNK + [blk1] * NCHUNK + [blk] * NCHUNK +
                 [blk, blk1, full((8, CW)),
                  full((H, 128)), full((8, 128)), full((128, 128)),
                  full((8, 128))],
        out_specs=pl.BlockSpec((8, 128), lambda i: (0, 0)),
        out_shape=jax.ShapeDtypeStruct((8, 128), jnp.float32),
        scratch_shapes=[pltpu.VMEM((BN, H), jnp.float32)],
    )(*args)


def kernel(x, edge_index, W1, b1, W2, b2, W3, b3, C1, cb1, C2, cb2):
    f32 = jnp.float32
    xp = jnp.zeros((NP, F), f32).at[:N].set(x)
    src = edge_index[0].reshape(NW, ET)
    dst = edge_index[1].reshape(NW, ET)
    srcp = jnp.pad(src, ((0, 0), (0, ETP - ET))).reshape(NW, NB, EB)
    dstp = jnp.pad(dst, ((0, 0), (0, ETP - ET)),
                   constant_values=TRASH).reshape(NW, NB, EB)
    ones_rows = jnp.ones((EB, CW), f32)
    zeros_rows = jnp.zeros((RPT, CW), f32)

    def pad_bias(b):
        return b.reshape(NCHUNK, CW)

    b1p, b2p, b3p = pad_bias(b1), pad_bias(b2), pad_bias(b3)
    c1t_pad = jnp.zeros((H, 128), f32).at[:, 0:64].set(C1.T)
    cb1_pad = jnp.zeros((8, 128), f32).at[0, 0:64].set(cb1)
    c2t_pad = jnp.zeros((128, 128), f32).at[0:64, 0:1].set(C2.T)
    cb2_pad = jnp.full((8, 128), cb2[0], f32)

    deg2 = _sc_degree(dstp, ones_rows, zeros_rows)
    p1 = _tc_first(xp, deg2, W1)
    a1 = _sc_aggregate(srcp, dstp, p1, zeros_rows)
    p2 = _tc_mid(a1, p1, deg2, b1p, W2)
    a2 = _sc_aggregate(srcp, dstp, p2, zeros_rows)
    p3 = _tc_mid(a2, p2, deg2, b2p, W3)
    a3 = _sc_aggregate(srcp, dstp, p3, zeros_rows)
    out = _tc_head(a3, p3, deg2, b3p, c1t_pad, cb1_pad, c2t_pad, cb2_pad)
    return out[0, 0].reshape(1)


# speedup vs baseline: 4.4650x; 1.0678x over previous
import functools

import jax
import jax.numpy as jnp
from jax import lax
from jax.experimental import pallas as pl
from jax.experimental.pallas import tpu as pltpu
from jax.experimental.pallas import tpu_sc as plsc

N = 10000
NP = 10240
F = 256
H = 512
CW = 128
NCHUNK = 4
NC = 2
NS = 16
NW = NC * NS
E = 160000
ET = E // NW
EB = 64
NBQ = 80
NBS = 40
ETP = EB * NBQ
NBUF = 4
NGRP = NBQ // NBUF
TRASH = N
RPT = NP // NS
BN = 512
G = NP // BN


def _sc_mesh():
    return plsc.VectorSubcoreMesh(core_axis_name="c", subcore_axis_name="s")


def _sc_degree(dstp, ones_rows, zeros_rows):
    @functools.partial(
        pl.kernel,
        mesh=_sc_mesh(),
        out_type=jax.ShapeDtypeStruct((NC * NP, CW), jnp.float32),
        scratch_types=[
            pltpu.VMEM((NBQ, EB), jnp.int32),
            pltpu.VMEM((EB, CW), jnp.float32),
            pltpu.VMEM_SHARED((NP, CW), jnp.float32),
            pltpu.SemaphoreType.DMA,
        ],
    )
    def k(dst_hbm, ones_hbm, zeros_hbm, out_hbm, dst_v, ones_v, acc, sem):
        cid = lax.axis_index("c")
        sid = lax.axis_index("s")
        w = cid * NS + sid
        pltpu.sync_copy(dst_hbm.at[w], dst_v)
        pltpu.sync_copy(ones_hbm, ones_v)

        pltpu.sync_copy(zeros_hbm, acc.at[pl.ds(sid * RPT, RPT)])
        plsc.subcore_barrier()

        def group(g, carry):
            for u in range(8):
                pltpu.async_copy(ones_v, acc.at[dst_v.at[g * 8 + u]], sem,
                                 add=True)
            for u in range(8):
                pltpu.make_async_copy(ones_v, acc.at[dst_v.at[g * 8 + u]],
                                      sem).wait()
            return carry

        lax.fori_loop(0, NBQ // 8, group, 0)
        plsc.subcore_barrier()
        pltpu.sync_copy(
            acc.at[pl.ds(sid * RPT, RPT)],
            out_hbm.at[pl.ds(cid * NP + sid * RPT, RPT)],
        )

    return k(dstp, ones_rows, zeros_rows)


def _sc_aggregate(srcp, dstp, tables, zeros_rows):
    @functools.partial(
        pl.kernel,
        mesh=_sc_mesh(),
        out_type=[jax.ShapeDtypeStruct((NC * NP, CW), jnp.float32)] * NCHUNK,
        scratch_types=[
            pltpu.VMEM((NBS, 128), jnp.int32),
            pltpu.VMEM((NBQ, EB), jnp.int32),
        ] + [pltpu.VMEM((EB, CW), jnp.float32)] * NBUF +
        [pltpu.VMEM_SHARED((NP, CW), jnp.float32)] +
        [pltpu.SemaphoreType.DMA] * (2 * NBUF + 1),
    )
    def k(src_hbm, dst_hbm, t0, t1, t2, t3, zeros_hbm,
          o0, o1, o2, o3,
          src_v, dst_v, gb0, gb1, gb2, gb3, acc,
          sg0, sg1, sg2, sg3, ss0, ss1, ss2, ss3, so):
        cid = lax.axis_index("c")
        sid = lax.axis_index("s")
        w = cid * NS + sid
        pltpu.sync_copy(src_hbm.at[w], src_v)
        pltpu.sync_copy(dst_hbm.at[w], dst_v)

        tabs = (t0, t1, t2, t3)
        outs = (o0, o1, o2, o3)
        bufs = (gb0, gb1, gb2, gb3)
        sgs = (sg0, sg1, sg2, sg3)
        sss = (ss0, ss1, ss2, ss3)
        my_rows = pl.ds(sid * RPT, RPT)

        def _sidx(j):
            return src_v.at[j // 2, pl.ds((j % 2) * EB, EB)]

        def gather(tab, j, b):
            pltpu.async_copy(tab.at[_sidx(j)], bufs[b], sgs[b])

        def gwait(tab, j, b):
            pltpu.make_async_copy(tab.at[_sidx(j)], bufs[b], sgs[b]).wait()

        def scat(j, b):
            pltpu.async_copy(bufs[b], acc.at[dst_v.at[j]], sss[b], add=True)

        def swait(j, b):
            pltpu.make_async_copy(bufs[b], acc.at[dst_v.at[j]], sss[b]).wait()

        pltpu.sync_copy(zeros_hbm, acc.at[my_rows])
        for b in range(NBUF):
            gather(tabs[0], b, b)
        plsc.subcore_barrier()

        for ci in range(len(tabs)):
            tab, out = tabs[ci], outs[ci]

            def group(jj, carry, tab=tab):
                base = jj * NBUF
                for b in range(NBUF):
                    gwait(tab, base + b, b)
                    scat(base + b, b)
                for b in range(NBUF):
                    swait(base + b, b)
                    gather(tab, base + NBUF + b, b)
                return carry

            lax.fori_loop(0, NGRP - 1, group, 0)
            base = (NGRP - 1) * NBUF
            for b in range(NBUF):
                gwait(tab, base + b, b)
                scat(base + b, b)
            for b in range(NBUF):
                swait(base + b, b)
            plsc.subcore_barrier()
            cp_out = pltpu.async_copy(
                acc.at[my_rows],
                out.at[pl.ds(cid * NP + sid * RPT, RPT)], so)
            if ci + 1 < len(tabs):
                for b in range(NBUF):
                    gather(tabs[ci + 1], b, b)
            cp_out.wait()
            if ci + 1 < len(tabs):
                pltpu.sync_copy(zeros_hbm, acc.at[my_rows])
                plsc.subcore_barrier()

    return k(srcp, dstp, *tables, zeros_rows)


def _dinv_cat(d0, d1):
    deg = d0 + d1 + 1.0
    dinv = lax.rsqrt(deg)
    return jnp.concatenate([dinv] * NCHUNK, axis=1)


def _split_store(pp, outs):
    for c, o in enumerate(outs):
        o[...] = pp[:, c * CW:(c + 1) * CW]


def _tc_first(xp, deg2, W1):
    def body(x_ref, d0_ref, d1_ref, w_ref, *outs):
        dinv8 = _dinv_cat(d0_ref[...], d1_ref[...])
        p = lax.dot_general(x_ref[...], w_ref[...], (((1,), (1,)), ((), ())),
                            preferred_element_type=jnp.float32)
        _split_store(p * dinv8, outs)

    return pl.pallas_call(
        body,
        grid=(G,),
        in_specs=[
            pl.BlockSpec((BN, F), lambda i: (i, 0)),
            pl.BlockSpec((BN, CW), lambda i: (i, 0)),
            pl.BlockSpec((BN, CW), lambda i: (i + G, 0)),
            pl.BlockSpec((H, F), lambda i: (0, 0)),
        ],
        out_specs=[pl.BlockSpec((BN, CW), lambda i: (i, 0))] * NCHUNK,
        out_shape=[jax.ShapeDtypeStruct((NP, CW), jnp.float32)] * NCHUNK,
    )(xp, deg2, deg2, W1)


def _relu_layer(a0s, a1s, ps, d0_ref, d1_ref, b_ref):
    dinv8 = _dinv_cat(d0_ref[...], d1_ref[...])
    cat = jnp.concatenate(
        [a0s[c][...] + a1s[c][...] + ps[c][...] for c in range(NCHUNK)], axis=1)
    brow = jnp.concatenate(
        [b_ref[c:c + 1, :] for c in range(NCHUNK)], axis=1)
    return jnp.maximum(dinv8 * cat + brow, 0.0), dinv8


def _tc_mid(aggs, ps, deg2, b_pad, W):
    def body(*refs):
        a0s, a1s, pcs = (refs[0:NCHUNK], refs[NCHUNK:2 * NCHUNK],
                         refs[2 * NCHUNK:3 * NCHUNK])
        d0_ref, d1_ref, b_ref, w_ref = refs[3 * NCHUNK:3 * NCHUNK + 4]
        outs = refs[3 * NCHUNK + 4:]
        h, dinv8 = _relu_layer(a0s, a1s, pcs, d0_ref, d1_ref, b_ref)
        p = lax.dot_general(h, w_ref[...], (((1,), (1,)), ((), ())),
                            preferred_element_type=jnp.float32)
        _split_store(p * dinv8, outs)

    blk = pl.BlockSpec((BN, CW), lambda i: (i, 0))
    blk1 = pl.BlockSpec((BN, CW), lambda i: (i + G, 0))
    args = list(aggs) + list(aggs) + list(ps) + [deg2, deg2, b_pad, W]
    return pl.pallas_call(
        body,
        grid=(G,),
        in_specs=[blk] * NCHUNK + [blk1] * NCHUNK + [blk] * NCHUNK +
                 [blk, blk1,
                  pl.BlockSpec((8, CW), lambda i: (0, 0)),
                  pl.BlockSpec((H, H), lambda i: (0, 0))],
        out_specs=[pl.BlockSpec((BN, CW), lambda i: (i, 0))] * NCHUNK,
        out_shape=[jax.ShapeDtypeStruct((NP, CW), jnp.float32)] * NCHUNK,
    )(*args)


def _tc_head(aggs, ps, deg2, b_pad, c1t_pad, cb1_pad, c2t_pad, cb2_pad):
    def body(*refs):
        a0s, a1s, pcs = (refs[0:NCHUNK], refs[NCHUNK:2 * NCHUNK],
                         refs[2 * NCHUNK:3 * NCHUNK])
        d0_ref, d1_ref, b_ref = refs[3 * NCHUNK:3 * NCHUNK + 3]
        c1t_ref, cb1_ref, c2t_ref, cb2_ref = refs[3 * NCHUNK + 3:3 * NCHUNK + 7]
        out_ref = refs[3 * NCHUNK + 7]
        acc = refs[3 * NCHUNK + 8]
        i = pl.program_id(0)
        h, _ = _relu_layer(a0s, a1s, pcs, d0_ref, d1_ref, b_ref)
        rows = i * BN + lax.broadcasted_iota(jnp.int32, (BN, 1), 0)
        h = jnp.where(rows < N, h, 0.0)

        @pl.when(i == 0)
        def _():
            acc[...] = jnp.zeros_like(acc)

        acc[...] += h

        @pl.when(i == G - 1)
        def _():
            g = jnp.sum(acc[...], axis=0, keepdims=True) * (1.0 / N)
            z = jnp.maximum(
                lax.dot_general(g, c1t_ref[...], (((1,), (0,)), ((), ())),
                                preferred_element_type=jnp.float32)
                + cb1_ref[0:1, :], 0.0)
            o = lax.dot_general(z, c2t_ref[...], (((1,), (0,)), ((), ())),
                                preferred_element_type=jnp.float32)
            o = o + cb2_ref[0:1, :]
            out_ref[...] = jnp.broadcast_to(o[0:1, 0:1], (8, 128))

    blk = pl.BlockSpec((BN, CW), lambda i: (i, 0))
    blk1 = pl.BlockSpec((BN, CW), lambda i: (i + G, 0))
    full = lambda shape: pl.BlockSpec(shape, lambda i: tuple(0 for _ in shape))
    args = list(aggs) + list(aggs) + list(ps) + [deg2, deg2, b_pad,
            c1t_pad, cb1_pad, c2t_pad, cb2_pad]
    return pl.pallas_call(
        body,
        grid=(G,),
        in_specs=[blk] * NCHUNK + [blk1] * NCHUNK + [blk] * NCHUNK +
                 [blk, blk1, full((8, CW)),
                  full((H, 128)), full((8, 128)), full((128, 128)),
                  full((8, 128))],
        out_specs=pl.BlockSpec((8, 128), lambda i: (0, 0)),
        out_shape=jax.ShapeDtypeStruct((8, 128), jnp.float32),
        scratch_shapes=[pltpu.VMEM((BN, H), jnp.float32)],
    )(*args)


def kernel(x, edge_index, W1, b1, W2, b2, W3, b3, C1, cb1, C2, cb2):
    f32 = jnp.float32
    xp = jnp.zeros((NP, F), f32).at[:N].set(x)
    src = edge_index[0].reshape(NW, ET)
    dst = edge_index[1].reshape(NW, ET)
    srcp = jnp.pad(src, ((0, 0), (0, ETP - ET))).reshape(NW, NBS, 128)
    dstp = jnp.pad(dst, ((0, 0), (0, ETP - ET)),
                   constant_values=TRASH).reshape(NW, NBQ, EB)
    ones_rows = jnp.ones((EB, CW), f32)
    zeros_rows = jnp.zeros((RPT, CW), f32)

    def pad_bias(b):
        return b.reshape(NCHUNK, CW)

    b1p, b2p, b3p = pad_bias(b1), pad_bias(b2), pad_bias(b3)
    c1t_pad = jnp.zeros((H, 128), f32).at[:, 0:64].set(C1.T)
    cb1_pad = jnp.zeros((8, 128), f32).at[0, 0:64].set(cb1)
    c2t_pad = jnp.zeros((128, 128), f32).at[0:64, 0:1].set(C2.T)
    cb2_pad = jnp.full((8, 128), cb2[0], f32)

    deg2 = _sc_degree(dstp, ones_rows, zeros_rows)
    p1 = _tc_first(xp, deg2, W1)
    a1 = _sc_aggregate(srcp, dstp, p1, zeros_rows)
    p2 = _tc_mid(a1, p1, deg2, b1p, W2)
    a2 = _sc_aggregate(srcp, dstp, p2, zeros_rows)
    p3 = _tc_mid(a2, p2, deg2, b2p, W3)
    a3 = _sc_aggregate(srcp, dstp, p3, zeros_rows)
    out = _tc_head(a3, p3, deg2, b3p, c1t_pad, cb1_pad, c2t_pad, cb2_pad)
    return out[0, 0].reshape(1)
